# baseline (speedup 1.0000x reference)
import functools

import jax
import jax.numpy as jnp
from jax import lax
from jax.experimental import pallas as pl
from jax.experimental.pallas import tpu as pltpu
from jax.experimental.pallas import tpu_sc as plsc

_NU = 50000
_NI = 50000
_NN = _NU + _NI
_D = 32
_NC, _NS = 2, 16
_HD = _D // _NC
_CHK = 256
_SUBW = 128
_NSUB = _CHK // _SUBW
_EVR = 2 * _NSUB
_NQ = 4
_ACC_PAD = 100096
_ZPT = _ACC_PAD // _NS


def _prop_body(cur, evm, wm, out, acc,
               rows0, rows1, rows2, rows3,
               evb0, evb1, evb2, evb3,
               wvb0, wvb1, wvb2, wvb3,
               isem, gsem, ssem):
    c = lax.axis_index("c")
    s = lax.axis_index("s")
    npt = evm.shape[1] // _EVR // _NS
    zero16 = jnp.zeros((16,), jnp.float32)
    rowsb = (rows0, rows1, rows2, rows3)
    evbb = (evb0, evb1, evb2, evb3)
    wvbb = (wvb0, wvb1, wvb2, wvb3)

    def _z(i, carry):
        rows0[i, 0:16] = zero16
        return carry
    lax.fori_loop(0, _CHK, _z, 0)

    def _zc(t, carry):
        pltpu.sync_copy(rows0, acc.at[pl.ds(s * _ZPT + t * _CHK, _CHK)])
        return carry
    lax.fori_loop(0, _ZPT // _CHK, _zc, 0)
    zrem = _ZPT % _CHK
    pltpu.sync_copy(rows0.at[pl.ds(0, zrem)],
                    acc.at[pl.ds(s * _ZPT + _ZPT - zrem, zrem)])
    plsc.subcore_barrier()

    gdn = lax.GatherDimensionNumbers(offset_dims=(), collapsed_slice_dims=(0,),
                                     start_index_map=(0,))

    def _step(g, carry):
        base = g * _NQ
        evds = []
        for q in range(_NQ):
            kq = base + q
            rb = (s * npt + kq) * _EVR
            wb = (s * npt + kq) * _NSUB
            evds.append(
                (pltpu.async_copy(evm.at[c, pl.ds(rb, _EVR)], evbb[q], isem),
                 pltpu.async_copy(wm.at[pl.ds(wb, _NSUB)], wvbb[q], isem)))
        gds = []
        for q in range(_NQ):
            for dsc in evds[q]:
                dsc.wait()
            gds.append([
                pltpu.async_copy(cur.at[evbb[q].at[j]],
                                 rowsb[q].at[pl.ds(j * _SUBW, _SUBW)], gsem)
                for j in range(_NSUB)])
        sds = []
        for q in range(_NQ):
            for dsc in gds[q]:
                dsc.wait()

            def _per4(r4, carry2, q=q):
                for u in range(8):
                    w16 = wvbb[q][r4, pl.ds(u * 16, 16)]
                    for i in range(16):
                        e = r4 * _SUBW + u * 16 + i
                        wb2 = lax.gather(
                            w16, jnp.full((16, 1), i, jnp.int32), gdn,
                            slice_sizes=(1,),
                            mode=lax.GatherScatterMode.PROMISE_IN_BOUNDS)
                        rowsb[q][e, 0:16] = rowsb[q][e, 0:16] * wb2
                return carry2
            lax.fori_loop(0, _NSUB, _per4, 0)

            sds.append([
                pltpu.async_copy(rowsb[q].at[pl.ds(j * _SUBW, _SUBW)],
                                 acc.at[evbb[q].at[_NSUB + j]], ssem,
                                 add=True)
                for j in range(_NSUB)])
        for q in range(_NQ):
            for dsc in sds[q]:
                dsc.wait()
        return carry
    lax.fori_loop(0, npt // _NQ, _step, 0)
    plsc.subcore_barrier()

    wb_l = s * _ZPT
    ob = c * _NN + wb_l
    nfull = 6160 // _CHK
    wrem = 6160 % _CHK

    def _wc(t, carry):
        pltpu.sync_copy(acc.at[pl.ds(wb_l + t * _CHK, _CHK)], rows0)
        pltpu.sync_copy(rows0, out.at[pl.ds(ob + t * _CHK, _CHK)])
        return carry
    lax.fori_loop(0, nfull, _wc, 0)
    pltpu.sync_copy(acc.at[pl.ds(wb_l + 6160 - wrem, wrem)],
                    rows0.at[pl.ds(0, wrem)])
    pltpu.sync_copy(rows0.at[pl.ds(0, wrem)],
                    out.at[pl.ds(ob + 6160 - wrem, wrem)])

    @pl.when(s < _NS - 1)
    def _tail():
        pltpu.sync_copy(acc.at[pl.ds(wb_l + 6160, 96)],
                        rows0.at[pl.ds(0, 96)])
        pltpu.sync_copy(rows0.at[pl.ds(0, 96)], out.at[pl.ds(ob + 6160, 96)])


def _users_body(t, idx, o, idxv, idxv2, rowsv, sem):
    c = lax.axis_index("c")
    s = lax.axis_index("s")
    wid = s * _NC + c
    per = idx.shape[0] // (_NC * _NS)
    base = wid * per
    pltpu.sync_copy(idx.at[pl.ds(base, per)], idxv)
    for q in range(per // 16):
        idxv2[pl.ds(q * 16, 16)] = idxv[pl.ds(q * 16, 16)] + _NN
    pltpu.async_copy(t.at[idxv], rowsv, sem).wait()
    pltpu.sync_copy(rowsv, o.at[pl.ds(base, per), pl.ds(0, _HD)])
    pltpu.async_copy(t.at[idxv2], rowsv, sem).wait()
    pltpu.sync_copy(rowsv, o.at[pl.ds(base, per), pl.ds(_HD, _HD)])


def _partial_body(u, it, out):
    dn = (((1,), (1,)), ((), ()))
    out[...] = lax.dot_general(u[...], it[...], dn,
                               preferred_element_type=jnp.float32)


def _scores_body(p, u, it, out):
    dn = (((1,), (1,)), ((), ()))
    acc = p[...] + lax.dot_general(u[...], it[...], dn,
                                   preferred_element_type=jnp.float32)
    out[...] = 1.0 / (1.0 + jnp.exp(-acc))


def kernel(user_index, user_table, item_table, edge_index, edge_weight):
    num_users, d = user_table.shape
    num_items = item_table.shape[0]
    n = num_users + num_items
    batch = user_index.shape[0]
    e = edge_index.shape[1]

    e_pad = -(-e // (_NS * _NQ * _CHK)) * (_NS * _NQ * _CHK)
    src = jnp.pad(edge_index[0].astype(jnp.int32), (0, e_pad - e))
    dst = jnp.pad(edge_index[1].astype(jnp.int32), (0, e_pad - e))
    w = jnp.pad(edge_weight, (0, e_pad - e))
    dstr = dst.reshape(-1, _NSUB, _SUBW)
    evm = jnp.stack([
        jnp.concatenate([src.reshape(-1, _NSUB, _SUBW), dstr], axis=1),
        jnp.concatenate([(src + n).reshape(-1, _NSUB, _SUBW), dstr], axis=1),
    ]).reshape(2, -1, _SUBW)
    wm = w.reshape(-1, _SUBW)

    con0 = jnp.concatenate(
        [user_table[:, :_HD], item_table[:, :_HD],
         user_table[:, _HD:], item_table[:, _HD:]], axis=0)

    mesh = plsc.VectorSubcoreMesh(core_axis_name="c", subcore_axis_name="s",
                                  num_cores=_NC, num_subcores=_NS)
    sc_params = pltpu.CompilerParams(use_tc_tiling_on_sc=False)
    prop = pl.kernel(
        _prop_body,
        out_type=jax.ShapeDtypeStruct((2 * n, _HD), jnp.float32),
        mesh=mesh,
        compiler_params=sc_params,
        scratch_types=(
            [pltpu.VMEM_SHARED((_ACC_PAD, _HD), jnp.float32)]
            + [pltpu.VMEM((_CHK, _HD), jnp.float32)] * _NQ
            + [pltpu.VMEM((_EVR, _SUBW), jnp.int32)] * _NQ
            + [pltpu.VMEM((_NSUB, _SUBW), jnp.float32)] * _NQ
            + [pltpu.SemaphoreType.DMA] * 3
        ),
    )
    per = batch // (_NC * _NS)
    ug = pl.kernel(
        _users_body,
        out_type=jax.ShapeDtypeStruct((batch, _D), jnp.float32),
        mesh=mesh,
        compiler_params=sc_params,
        scratch_types=[
            pltpu.VMEM((per,), jnp.int32),
            pltpu.VMEM((per,), jnp.int32),
            pltpu.VMEM((per, _HD), jnp.float32),
            pltpu.SemaphoreType.DMA,
        ],
    )
    uidx = user_index.astype(jnp.int32)

    lam = [1.0 / 9.0, 4.0 / 9.0, 1.0]
    bi = 2048
    gi = -(-num_items // bi)

    h1 = prop(con0, evm, wm)
    u0 = ug(con0, uidx)
    u1 = ug(h1, uidx)
    wm2 = wm + (u0[0, 0] + u1[0, 0]) * 0.0
    h2 = prop(h1, evm, wm2)

    u01 = jnp.concatenate([u0 * lam[0], u1 * lam[1]],
                          axis=1).astype(jnp.bfloat16)
    it01 = jnp.concatenate(
        [con0[num_users:n], con0[n + num_users:],
         h1[num_users:n], h1[n + num_users:]], axis=1).astype(jnp.bfloat16)
    p01 = pl.pallas_call(
        _partial_body,
        grid=(gi,),
        in_specs=[pl.BlockSpec((batch, 2 * _D), lambda j: (0, 0)),
                  pl.BlockSpec((bi, 2 * _D), lambda j: (j, 0))],
        out_specs=pl.BlockSpec((batch, bi), lambda j: (0, j)),
        out_shape=jax.ShapeDtypeStruct((batch, num_items), jnp.float32),
    )(u01, it01)

    u2 = ug(h2, uidx)
    u2b = (u2 * lam[2]).astype(jnp.bfloat16)
    it2 = jnp.concatenate(
        [h2[num_users:n], h2[n + num_users:]], axis=1).astype(jnp.bfloat16)
    scores = pl.pallas_call(
        _scores_body,
        grid=(gi,),
        in_specs=[pl.BlockSpec((batch, bi), lambda j: (0, j)),
                  pl.BlockSpec((batch, _D), lambda j: (0, 0)),
                  pl.BlockSpec((bi, _D), lambda j: (j, 0))],
        out_specs=pl.BlockSpec((batch, bi), lambda j: (0, j)),
        out_shape=jax.ShapeDtypeStruct((batch, num_items), jnp.float32),
    )(p01, u2b, it2)
    return scores


# speedup vs baseline: 1.0007x; 1.0007x over previous
import functools

import jax
import jax.numpy as jnp
from jax import lax
from jax.experimental import pallas as pl
from jax.experimental.pallas import tpu as pltpu
from jax.experimental.pallas import tpu_sc as plsc

_NU = 50000
_NI = 50000
_NN = _NU + _NI
_D = 32
_NC, _NS = 2, 16
_HD = _D // _NC
_CHK = 256
_SUBW = 128
_NSUB = _CHK // _SUBW
_EVR = 2 * _NSUB
_NQ = 4
_ACC_PAD = 100096
_ZPT = _ACC_PAD // _NS


def _prop_body(cur, evm, wm, out, acc,
               rows0, rows1, rows2, rows3,
               evb0, evb1, evb2, evb3,
               wvb0, wvb1, wvb2, wvb3,
               isem, gsem, ssem):
    c = lax.axis_index("c")
    s = lax.axis_index("s")
    npt = evm.shape[1] // _EVR // _NS
    zero16 = jnp.zeros((16,), jnp.float32)
    rowsb = (rows0, rows1, rows2, rows3)
    evbb = (evb0, evb1, evb2, evb3)
    wvbb = (wvb0, wvb1, wvb2, wvb3)

    def _z(i, carry):
        rows0[i, 0:16] = zero16
        return carry
    lax.fori_loop(0, _CHK, _z, 0)

    def _zc(t, carry):
        pltpu.sync_copy(rows0, acc.at[pl.ds(s * _ZPT + t * _CHK, _CHK)])
        return carry
    lax.fori_loop(0, _ZPT // _CHK, _zc, 0)
    zrem = _ZPT % _CHK
    pltpu.sync_copy(rows0.at[pl.ds(0, zrem)],
                    acc.at[pl.ds(s * _ZPT + _ZPT - zrem, zrem)])
    plsc.subcore_barrier()

    gdn = lax.GatherDimensionNumbers(offset_dims=(), collapsed_slice_dims=(0,),
                                     start_index_map=(0,))

    def _step(g, carry):
        base = g * _NQ
        evds = []
        for q in range(_NQ):
            kq = base + q
            rb = (s * npt + kq) * _EVR
            wb = (s * npt + kq) * _NSUB
            evds.append(
                (pltpu.async_copy(evm.at[c, pl.ds(rb, _EVR)], evbb[q], isem),
                 pltpu.async_copy(wm.at[pl.ds(wb, _NSUB)], wvbb[q], isem)))
        gds = []
        for q in range(_NQ):
            for dsc in evds[q]:
                dsc.wait()
            gds.append([
                pltpu.async_copy(cur.at[evbb[q].at[j]],
                                 rowsb[q].at[pl.ds(j * _SUBW, _SUBW)], gsem)
                for j in range(_NSUB)])
        sds = []
        for q in range(_NQ):
            for dsc in gds[q]:
                dsc.wait()

            def _per4(r4, carry2, q=q):
                for u in range(8):
                    w16 = wvbb[q][r4, pl.ds(u * 16, 16)]
                    for i in range(16):
                        e = r4 * _SUBW + u * 16 + i
                        wb2 = lax.gather(
                            w16, jnp.full((16, 1), i, jnp.int32), gdn,
                            slice_sizes=(1,),
                            mode=lax.GatherScatterMode.PROMISE_IN_BOUNDS)
                        rowsb[q][e, 0:16] = rowsb[q][e, 0:16] * wb2
                return carry2
            lax.fori_loop(0, _NSUB, _per4, 0)

            sds.append([
                pltpu.async_copy(rowsb[q].at[pl.ds(j * _SUBW, _SUBW)],
                                 acc.at[evbb[q].at[_NSUB + j]], ssem,
                                 add=True)
                for j in range(_NSUB)])
        for q in range(_NQ):
            for dsc in sds[q]:
                dsc.wait()
        return carry
    lax.fori_loop(0, npt // _NQ, _step, 0)
    plsc.subcore_barrier()

    wb_l = s * _ZPT
    ob = c * _NN + wb_l
    nfull = 6160 // _CHK
    wrem = 6160 % _CHK

    def _wc(t, carry):
        pltpu.sync_copy(acc.at[pl.ds(wb_l + t * _CHK, _CHK)], rows0)
        pltpu.sync_copy(rows0, out.at[pl.ds(ob + t * _CHK, _CHK)])
        return carry
    lax.fori_loop(0, nfull, _wc, 0)
    pltpu.sync_copy(acc.at[pl.ds(wb_l + 6160 - wrem, wrem)],
                    rows0.at[pl.ds(0, wrem)])
    pltpu.sync_copy(rows0.at[pl.ds(0, wrem)],
                    out.at[pl.ds(ob + 6160 - wrem, wrem)])

    @pl.when(s < _NS - 1)
    def _tail():
        pltpu.sync_copy(acc.at[pl.ds(wb_l + 6160, 96)],
                        rows0.at[pl.ds(0, 96)])
        pltpu.sync_copy(rows0.at[pl.ds(0, 96)], out.at[pl.ds(ob + 6160, 96)])


def _users_body(t, idx, o, idxv, idxv2, rowsv, sem):
    c = lax.axis_index("c")
    s = lax.axis_index("s")
    wid = s * _NC + c
    per = idx.shape[0] // (_NC * _NS)
    base = wid * per
    pltpu.sync_copy(idx.at[pl.ds(base, per)], idxv)
    for q in range(per // 16):
        idxv2[pl.ds(q * 16, 16)] = idxv[pl.ds(q * 16, 16)] + _NN
    pltpu.async_copy(t.at[idxv], rowsv, sem).wait()
    pltpu.sync_copy(rowsv, o.at[pl.ds(base, per), pl.ds(0, _HD)])
    pltpu.async_copy(t.at[idxv2], rowsv, sem).wait()
    pltpu.sync_copy(rowsv, o.at[pl.ds(base, per), pl.ds(_HD, _HD)])


def _partial_body(u, it, out):
    dn = (((1,), (1,)), ((), ()))
    out[...] = lax.dot_general(u[...], it[...], dn,
                               preferred_element_type=jnp.float32)


def _scores_body(p, u, it, out):
    dn = (((1,), (1,)), ((), ()))
    acc = p[...] + lax.dot_general(u[...], it[...], dn,
                                   preferred_element_type=jnp.float32)
    out[...] = 1.0 / (1.0 + jnp.exp(-acc))


def kernel(user_index, user_table, item_table, edge_index, edge_weight):
    num_users, d = user_table.shape
    num_items = item_table.shape[0]
    n = num_users + num_items
    batch = user_index.shape[0]
    e = edge_index.shape[1]

    e_pad = -(-e // (_NS * _NQ * _CHK)) * (_NS * _NQ * _CHK)
    src = jnp.pad(edge_index[0].astype(jnp.int32), (0, e_pad - e))
    dst = jnp.pad(edge_index[1].astype(jnp.int32), (0, e_pad - e))
    w = jnp.pad(edge_weight, (0, e_pad - e))
    dstr = dst.reshape(-1, _NSUB, _SUBW)
    evm = jnp.stack([
        jnp.concatenate([src.reshape(-1, _NSUB, _SUBW), dstr], axis=1),
        jnp.concatenate([(src + n).reshape(-1, _NSUB, _SUBW), dstr], axis=1),
    ]).reshape(2, -1, _SUBW)
    wm = w.reshape(-1, _SUBW)

    con0 = jnp.concatenate(
        [user_table[:, :_HD], item_table[:, :_HD],
         user_table[:, _HD:], item_table[:, _HD:]], axis=0)

    mesh = plsc.VectorSubcoreMesh(core_axis_name="c", subcore_axis_name="s",
                                  num_cores=_NC, num_subcores=_NS)
    sc_params = pltpu.CompilerParams(use_tc_tiling_on_sc=False)
    prop = pl.kernel(
        _prop_body,
        out_type=jax.ShapeDtypeStruct((2 * n, _HD), jnp.float32),
        mesh=mesh,
        compiler_params=sc_params,
        scratch_types=(
            [pltpu.VMEM_SHARED((_ACC_PAD, _HD), jnp.float32)]
            + [pltpu.VMEM((_CHK, _HD), jnp.float32)] * _NQ
            + [pltpu.VMEM((_EVR, _SUBW), jnp.int32)] * _NQ
            + [pltpu.VMEM((_NSUB, _SUBW), jnp.float32)] * _NQ
            + [pltpu.SemaphoreType.DMA] * 3
        ),
    )
    per = batch // (_NC * _NS)
    ug = pl.kernel(
        _users_body,
        out_type=jax.ShapeDtypeStruct((batch, _D), jnp.float32),
        mesh=mesh,
        compiler_params=sc_params,
        scratch_types=[
            pltpu.VMEM((per,), jnp.int32),
            pltpu.VMEM((per,), jnp.int32),
            pltpu.VMEM((per, _HD), jnp.float32),
            pltpu.SemaphoreType.DMA,
        ],
    )
    uidx = user_index.astype(jnp.int32)

    lam = [1.0 / 9.0, 4.0 / 9.0, 1.0]
    bi = 2048
    gi = -(-num_items // bi)

    h1 = prop(con0, evm, wm)
    u0 = ug(con0, uidx)
    u1 = ug(h1, uidx)
    wm2 = wm + (u0[0, 0] + u1[0, 0]) * 0.0
    h2 = prop(h1, evm, wm2)

    u01 = jnp.concatenate([u0 * lam[0], u1 * lam[1]],
                          axis=1).astype(jnp.bfloat16)
    it01 = jnp.concatenate(
        [con0[num_users:n], con0[n + num_users:],
         h1[num_users:n], h1[n + num_users:]], axis=1).astype(jnp.bfloat16)
    p01 = pl.pallas_call(
        _partial_body,
        grid=(gi,),
        in_specs=[pl.BlockSpec((batch, 2 * _D), lambda j: (0, 0)),
                  pl.BlockSpec((bi, 2 * _D), lambda j: (j, 0))],
        out_specs=pl.BlockSpec((batch, bi), lambda j: (0, j)),
        out_shape=jax.ShapeDtypeStruct((batch, num_items), jnp.float32),
    )(u01, it01)

    u2 = ug(h2, uidx)
    u2b = (u2 * lam[2]).astype(jnp.bfloat16)
    it2 = jnp.concatenate(
        [h2[num_users:n], h2[n + num_users:]], axis=1).astype(jnp.bfloat16)
    scores = pl.pallas_call(
        _scores_body,
        grid=(gi,),
        in_specs=[pl.BlockSpec((batch, bi), lambda j: (0, j)),
                  pl.BlockSpec((batch, _D), lambda j: (0, 0)),
                  pl.BlockSpec((bi, _D), lambda j: (j, 0))],
        out_specs=pl.BlockSpec((batch, bi), lambda j: (0, j)),
        out_shape=jax.ShapeDtypeStruct((batch, num_items), jnp.float32),
        input_output_aliases={0: 0},
    )(p01, u2b, it2)
    return scores


# speedup vs baseline: 1.0642x; 1.0634x over previous
import functools

import jax
import jax.numpy as jnp
from jax import lax
from jax.experimental import pallas as pl
from jax.experimental.pallas import tpu as pltpu
from jax.experimental.pallas import tpu_sc as plsc

_NU = 50000
_NI = 50000
_NN = _NU + _NI
_D = 32
_NC, _NS = 2, 16
_HD = _D // _NC
_CHK = 256
_SUBW = 128
_NSUB = _CHK // _SUBW
_EVR = 2 * _NSUB
_NQ = 4
_ACC_PAD = 100096
_ZPT = _ACC_PAD // _NS


def _prop_body(cur, evm, wm, out, acc,
               rows0, rows1, rows2, rows3,
               evb0, evb1, evb2, evb3,
               wvb0, wvb1, wvb2, wvb3,
               isem, gsem, ssem):
    c = lax.axis_index("c")
    s = lax.axis_index("s")
    npt = evm.shape[1] // _EVR // _NS
    zero16 = jnp.zeros((16,), jnp.float32)
    rowsb = (rows0, rows1, rows2, rows3)
    evbb = (evb0, evb1, evb2, evb3)
    wvbb = (wvb0, wvb1, wvb2, wvb3)

    def _z(i, carry):
        rows0[i, 0:16] = zero16
        return carry
    lax.fori_loop(0, _CHK, _z, 0)

    def _zc(t, carry):
        pltpu.sync_copy(rows0, acc.at[pl.ds(s * _ZPT + t * _CHK, _CHK)])
        return carry
    lax.fori_loop(0, _ZPT // _CHK, _zc, 0)
    zrem = _ZPT % _CHK
    pltpu.sync_copy(rows0.at[pl.ds(0, zrem)],
                    acc.at[pl.ds(s * _ZPT + _ZPT - zrem, zrem)])
    plsc.subcore_barrier()

    gdn = lax.GatherDimensionNumbers(offset_dims=(), collapsed_slice_dims=(0,),
                                     start_index_map=(0,))

    def _step(g, carry):
        base = g * _NQ
        evds = []
        for q in range(_NQ):
            kq = base + q
            rb = (s * npt + kq) * _EVR
            wb = (s * npt + kq) * _NSUB
            evds.append(
                (pltpu.async_copy(evm.at[c, pl.ds(rb, _EVR)], evbb[q], isem),
                 pltpu.async_copy(wm.at[pl.ds(wb, _NSUB)], wvbb[q], isem)))
        gds = []
        for q in range(_NQ):
            for dsc in evds[q]:
                dsc.wait()
            gds.append([
                pltpu.async_copy(cur.at[evbb[q].at[j]],
                                 rowsb[q].at[pl.ds(j * _SUBW, _SUBW)], gsem)
                for j in range(_NSUB)])
        sds = []
        for q in range(_NQ):
            for dsc in gds[q]:
                dsc.wait()

            def _per4(r4, carry2, q=q):
                for u in range(8):
                    w16 = wvbb[q][r4, pl.ds(u * 16, 16)]
                    for i in range(16):
                        e = r4 * _SUBW + u * 16 + i
                        wb2 = lax.gather(
                            w16, jnp.full((16, 1), i, jnp.int32), gdn,
                            slice_sizes=(1,),
                            mode=lax.GatherScatterMode.PROMISE_IN_BOUNDS)
                        rowsb[q][e, 0:16] = rowsb[q][e, 0:16] * wb2
                return carry2
            lax.fori_loop(0, _NSUB, _per4, 0)

            sds.append([
                pltpu.async_copy(rowsb[q].at[pl.ds(j * _SUBW, _SUBW)],
                                 acc.at[evbb[q].at[_NSUB + j]], ssem,
                                 add=True)
                for j in range(_NSUB)])
        for q in range(_NQ):
            for dsc in sds[q]:
                dsc.wait()
        return carry
    lax.fori_loop(0, npt // _NQ, _step, 0)
    plsc.subcore_barrier()

    wb_l = s * _ZPT
    ob = c * _NN + wb_l
    nfull = 6160 // _CHK
    wrem = 6160 % _CHK

    def _wc(t, carry):
        pltpu.sync_copy(acc.at[pl.ds(wb_l + t * _CHK, _CHK)], rows0)
        pltpu.sync_copy(rows0, out.at[pl.ds(ob + t * _CHK, _CHK)])
        return carry
    lax.fori_loop(0, nfull, _wc, 0)
    pltpu.sync_copy(acc.at[pl.ds(wb_l + 6160 - wrem, wrem)],
                    rows0.at[pl.ds(0, wrem)])
    pltpu.sync_copy(rows0.at[pl.ds(0, wrem)],
                    out.at[pl.ds(ob + 6160 - wrem, wrem)])

    @pl.when(s < _NS - 1)
    def _tail():
        pltpu.sync_copy(acc.at[pl.ds(wb_l + 6160, 96)],
                        rows0.at[pl.ds(0, 96)])
        pltpu.sync_copy(rows0.at[pl.ds(0, 96)], out.at[pl.ds(ob + 6160, 96)])


def _users_body(t0, t1, t2, idx, o0, o1, o2, idxv, idxv2, rowsv, sem):
    c = lax.axis_index("c")
    s = lax.axis_index("s")
    wid = s * _NC + c
    per = idx.shape[0] // (_NC * _NS)
    base = wid * per
    pltpu.sync_copy(idx.at[pl.ds(base, per)], idxv)
    for q in range(per // 16):
        idxv2[pl.ds(q * 16, 16)] = idxv[pl.ds(q * 16, 16)] + _NN
    for t, o in ((t0, o0), (t1, o1), (t2, o2)):
        pltpu.async_copy(t.at[idxv], rowsv, sem).wait()
        pltpu.sync_copy(rowsv, o.at[pl.ds(base, per), pl.ds(0, _HD)])
        pltpu.async_copy(t.at[idxv2], rowsv, sem).wait()
        pltpu.sync_copy(rowsv, o.at[pl.ds(base, per), pl.ds(_HD, _HD)])


def _scores_body(u, it, out):
    dn = (((1,), (1,)), ((), ()))
    acc = lax.dot_general(u[...], it[...], dn,
                          preferred_element_type=jnp.float32)
    out[...] = 1.0 / (1.0 + jnp.exp(-acc))


def kernel(user_index, user_table, item_table, edge_index, edge_weight):
    num_users, d = user_table.shape
    num_items = item_table.shape[0]
    n = num_users + num_items
    batch = user_index.shape[0]
    e = edge_index.shape[1]

    e_pad = -(-e // (_NS * _NQ * _CHK)) * (_NS * _NQ * _CHK)
    src = jnp.pad(edge_index[0].astype(jnp.int32), (0, e_pad - e))
    dst = jnp.pad(edge_index[1].astype(jnp.int32), (0, e_pad - e))
    w = jnp.pad(edge_weight, (0, e_pad - e))
    dstr = dst.reshape(-1, _NSUB, _SUBW)
    evm = jnp.stack([
        jnp.concatenate([src.reshape(-1, _NSUB, _SUBW), dstr], axis=1),
        jnp.concatenate([(src + n).reshape(-1, _NSUB, _SUBW), dstr], axis=1),
    ]).reshape(2, -1, _SUBW)
    wm = w.reshape(-1, _SUBW)

    con0 = jnp.concatenate(
        [user_table[:, :_HD], item_table[:, :_HD],
         user_table[:, _HD:], item_table[:, _HD:]], axis=0)

    mesh = plsc.VectorSubcoreMesh(core_axis_name="c", subcore_axis_name="s",
                                  num_cores=_NC, num_subcores=_NS)
    sc_params = pltpu.CompilerParams(use_tc_tiling_on_sc=False)
    prop = pl.kernel(
        _prop_body,
        out_type=jax.ShapeDtypeStruct((2 * n, _HD), jnp.float32),
        mesh=mesh,
        compiler_params=sc_params,
        scratch_types=(
            [pltpu.VMEM_SHARED((_ACC_PAD, _HD), jnp.float32)]
            + [pltpu.VMEM((_CHK, _HD), jnp.float32)] * _NQ
            + [pltpu.VMEM((_EVR, _SUBW), jnp.int32)] * _NQ
            + [pltpu.VMEM((_NSUB, _SUBW), jnp.float32)] * _NQ
            + [pltpu.SemaphoreType.DMA] * 3
        ),
    )
    h1 = prop(con0, evm, wm)
    h2 = prop(h1, evm, wm)

    per = batch // (_NC * _NS)
    ug = pl.kernel(
        _users_body,
        out_type=[jax.ShapeDtypeStruct((batch, _D), jnp.float32)] * 3,
        mesh=mesh,
        compiler_params=sc_params,
        scratch_types=[
            pltpu.VMEM((per,), jnp.int32),
            pltpu.VMEM((per,), jnp.int32),
            pltpu.VMEM((per, _HD), jnp.float32),
            pltpu.SemaphoreType.DMA,
        ],
    )
    u0, u1, u2 = ug(con0, h1, h2, user_index.astype(jnp.int32))

    lam = [1.0 / 9.0, 4.0 / 9.0, 1.0]
    u = jnp.concatenate([u0 * lam[0], u1 * lam[1], u2 * lam[2]],
                        axis=1).astype(jnp.bfloat16)
    it = jnp.concatenate(
        [con0[num_users:n], con0[n + num_users:],
         h1[num_users:n], h1[n + num_users:],
         h2[num_users:n], h2[n + num_users:]], axis=1).astype(jnp.bfloat16)
    bi = 2048
    gi = -(-num_items // bi)
    scores = pl.pallas_call(
        _scores_body,
        grid=(gi,),
        in_specs=[pl.BlockSpec((batch, 3 * _D), lambda j: (0, 0)),
                  pl.BlockSpec((bi, 3 * _D), lambda j: (j, 0))],
        out_specs=pl.BlockSpec((batch, bi), lambda j: (0, j)),
        out_shape=jax.ShapeDtypeStruct((batch, num_items), jnp.float32),
    )(u, it)
    return scores
